# baseline (speedup 1.0000x reference)
import functools

import jax
import jax.numpy as jnp
import numpy as np
from jax import lax
from jax.experimental import pallas as pl
from jax.experimental.pallas import tpu as pltpu
from jax.experimental.pallas import tpu_sc as plsc

N = 10000
NUM_MOL = 100
HIDDEN = 128
FILTERS = 128
NUM_INT = 6
NUM_G = 50
CUTOFF = 10.0
MAXNB = 32

BN = 200
NBLK = N // BN
RB = 128
NPAD = 10240
RNB = NPAD // RB
W = 512
EB = BN * MAXNB
E = N * MAXNB

_OFFS = np.linspace(0.0, CUTOFF, NUM_G, dtype=np.float32)
_COEFF = np.float32(-0.5) / (_OFFS[1] - _OFFS[0]) ** 2
_STEP = np.float32(CUTOFF / (NUM_G - 1))
_LOG2 = np.float32(np.log(2.0))
_PI = np.float32(np.pi)


def _ssp(x):
    return jnp.maximum(x, 0.0) + jnp.log(1.0 + jnp.exp(-jnp.abs(x))) - _LOG2


def _radius_body(ws_ref, posT_ref, sqr_ref, brow_ref, pos_ref, sqc_ref,
                 bcol_ref, src_ref, dist_ref, c_ref):
    i = pl.program_id(0)
    ws = pl.multiple_of(ws_ref[0, 0], 128)
    pcT = posT_ref[...]
    posw = pos_ref[pl.ds(ws, W), :]
    sqw = sqc_ref[pl.ds(ws, W), :]
    bw = bcol_ref[pl.ds(ws, W), :]
    br = brow_ref[...]
    sqr = sqr_ref[...]
    d2 = sqw + sqr - 2.0 * jnp.dot(posw, pcT, preferred_element_type=jnp.float32)
    d2 = jnp.maximum(d2, 0.0)

    row_id = i * RB + lax.broadcasted_iota(jnp.int32, (1, RB), 1)
    col_id = ws + lax.broadcasted_iota(jnp.int32, (W, 1), 0)
    valid = (bw == br) & (col_id != row_id) & (d2 <= CUTOFF * CUTOFF)
    d2m = jnp.where(valid, d2, jnp.inf)

    ils = lax.broadcasted_iota(jnp.int32, (W, 1), 0).astype(jnp.float32)
    src_rows = []
    d2_rows = []
    for _ in range(MAXNB):
        m = jnp.min(d2m, axis=0, keepdims=True)
        eq = d2m == m
        jloc = jnp.min(jnp.where(eq, ils, np.float32(1e9)), axis=0,
                       keepdims=True)
        src_rows.append(ws + jloc.astype(jnp.int32))
        d2_rows.append(m)
        d2m = jnp.where(ils == jloc, jnp.inf, d2m)

    src_blk = jnp.concatenate(src_rows, axis=0)
    d2_blk = jnp.concatenate(d2_rows, axis=0)
    maskf = (d2_blk <= CUTOFF * CUTOFF).astype(jnp.float32)
    dist = jnp.sqrt(jnp.where(d2_blk <= CUTOFF * CUTOFF, d2_blk, 1.0))
    cfac = 0.5 * (jnp.cos(dist * _PI / CUTOFF) + 1.0) * maskf
    src_ref[...] = src_blk
    dist_ref[...] = dist
    c_ref[...] = cfac


def _radius_graph(ws3, posT_pad, sq_row, batch_row, pos_pad, sq_col,
                  batch_col):
    return pl.pallas_call(
        _radius_body,
        grid=(RNB,),
        in_specs=[
            pl.BlockSpec((None, 1, 1), lambda i: (i, 0, 0)),
            pl.BlockSpec((3, RB), lambda i: (0, i)),
            pl.BlockSpec((1, RB), lambda i: (0, i)),
            pl.BlockSpec((1, RB), lambda i: (0, i)),
            pl.BlockSpec((NPAD, 3), lambda i: (0, 0)),
            pl.BlockSpec((NPAD, 1), lambda i: (0, 0)),
            pl.BlockSpec((NPAD, 1), lambda i: (0, 0)),
        ],
        out_specs=[
            pl.BlockSpec((MAXNB, RB), lambda i: (0, i)),
            pl.BlockSpec((MAXNB, RB), lambda i: (0, i)),
            pl.BlockSpec((MAXNB, RB), lambda i: (0, i)),
        ],
        out_shape=[
            jax.ShapeDtypeStruct((MAXNB, NPAD), jnp.int32),
            jax.ShapeDtypeStruct((MAXNB, NPAD), jnp.float32),
            jax.ShapeDtypeStruct((MAXNB, NPAD), jnp.float32),
        ],
    )(ws3, posT_pad, sq_row, batch_row, pos_pad, sq_col, batch_col)


def _embed_body(z_ref, emb_ref, lin1_ref, h_ref, xs_ref):
    zb = z_ref[...]
    ids = lax.broadcasted_iota(jnp.int32, (1, 100), 1)
    oh = (zb == ids).astype(jnp.float32)
    h = jnp.dot(oh, emb_ref[...], preferred_element_type=jnp.float32)
    h_ref[...] = h
    xs_ref[...] = jnp.dot(h, lin1_ref[...], preferred_element_type=jnp.float32)


def _embed(z2, emb, lin1_0):
    return pl.pallas_call(
        _embed_body,
        grid=(NBLK,),
        in_specs=[
            pl.BlockSpec((BN, 1), lambda i: (i, 0)),
            pl.BlockSpec((100, HIDDEN), lambda i: (0, 0)),
            pl.BlockSpec((HIDDEN, FILTERS), lambda i: (0, 0)),
        ],
        out_specs=[
            pl.BlockSpec((BN, HIDDEN), lambda i: (i, 0)),
            pl.BlockSpec((BN, FILTERS), lambda i: (i, 0)),
        ],
        out_shape=[
            jax.ShapeDtypeStruct((N, HIDDEN), jnp.float32),
            jax.ShapeDtypeStruct((N, FILTERS), jnp.float32),
        ],
    )(z2, emb, lin1_0)


_NW = 32
_BPW = E // _NW
_CH = 200
_NCH = _BPW // _CH


def _sc_gather(xs, idx):
    mesh = plsc.VectorSubcoreMesh(core_axis_name="c", subcore_axis_name="s")

    @functools.partial(
        pl.kernel,
        out_type=jax.ShapeDtypeStruct((E, HIDDEN), jnp.float32),
        mesh=mesh,
        scratch_types=[
            pltpu.VMEM((_BPW,), jnp.int32),
            pltpu.VMEM((2, _CH, HIDDEN), jnp.float32),
            pltpu.SemaphoreType.DMA,
            pltpu.SemaphoreType.DMA,
            pltpu.SemaphoreType.DMA,
            pltpu.SemaphoreType.DMA,
        ],
    )
    def k(table_hbm, idx_hbm, out_hbm, idx_v, rows_v, sg0, sg1, so0, so1):
        wid = lax.axis_index("c") * 16 + lax.axis_index("s")
        base0 = wid * _BPW
        sg = (sg0, sg1)
        so = (so0, so1)

        def gather_start(c, b):
            pltpu.async_copy(table_hbm.at[idx_v.at[pl.ds(c * _CH, _CH)]],
                             rows_v.at[b], sg[b])

        def gather_wait(c, b):
            pltpu.make_async_copy(table_hbm.at[idx_v.at[pl.ds(c * _CH, _CH)]],
                                  rows_v.at[b], sg[b]).wait()

        def out_start(c, b):
            pltpu.async_copy(rows_v.at[b],
                             out_hbm.at[pl.ds(base0 + c * _CH, _CH)], so[b])

        def out_wait(c, b):
            pltpu.make_async_copy(rows_v.at[b],
                                  out_hbm.at[pl.ds(base0 + c * _CH, _CH)],
                                  so[b]).wait()

        pltpu.sync_copy(idx_hbm.at[pl.ds(base0, _BPW)], idx_v)
        gather_start(0, 0)
        gather_wait(0, 0)
        out_start(0, 0)
        gather_start(1, 1)

        @pl.loop(1, _NCH - 1, step=2)
        def _(c):
            gather_wait(c, 1)
            out_start(c, 1)
            out_wait(c - 1, 0)
            gather_start(c + 1, 0)
            gather_wait(c + 1, 0)
            out_start(c + 1, 0)
            out_wait(c, 1)
            gather_start(c + 2, 1)

        c = _NCH - 1
        gather_wait(c, 1)
        out_start(c, 1)
        out_wait(c - 1, 0)
        out_wait(c, 1)

    return k(xs, idx)


def _interact_body(g_ref, d_ref, c_ref, h_ref,
                   w1_ref, b1_ref, w2_ref, b2_ref,
                   l2w_ref, l2b_ref, lw_ref, lb_ref, lin1n_ref,
                   h_out, xs_out):
    d3 = d_ref[...][:, :, None]
    cf3 = c_ref[...][:, :, None]
    offs = lax.broadcasted_iota(
        jnp.int32, (1, 1, NUM_G), 2).astype(jnp.float32) * _STEP
    rbf = jnp.exp(_COEFF * (d3 - offs) ** 2).reshape(EB, NUM_G)
    t = jnp.dot(rbf, w1_ref[...], preferred_element_type=jnp.float32) + b1_ref[...]
    t = _ssp(t)
    wf = jnp.dot(t, w2_ref[...], preferred_element_type=jnp.float32) + b2_ref[...]
    wf3 = wf.reshape(BN, MAXNB, FILTERS) * cf3
    msg = g_ref[...].reshape(BN, MAXNB, FILTERS) * wf3
    agg = jnp.sum(msg, axis=1)
    v = jnp.dot(agg, l2w_ref[...], preferred_element_type=jnp.float32) + l2b_ref[...]
    v = _ssp(v)
    v = jnp.dot(v, lw_ref[...], preferred_element_type=jnp.float32) + lb_ref[...]
    hn = h_ref[...] + v
    h_out[...] = hn
    xs_out[...] = jnp.dot(hn, lin1n_ref[...],
                          preferred_element_type=jnp.float32)


def _interact(g, dist, cfac, h, w1, b1, w2, b2, l2w, l2b, lw, lb, lin1n):
    full = lambda a, b: pl.BlockSpec((a, b), lambda i: (0, 0))
    return pl.pallas_call(
        _interact_body,
        grid=(NBLK,),
        in_specs=[
            pl.BlockSpec((EB, HIDDEN), lambda i: (i, 0)),
            pl.BlockSpec((BN, MAXNB), lambda i: (i, 0)),
            pl.BlockSpec((BN, MAXNB), lambda i: (i, 0)),
            pl.BlockSpec((BN, HIDDEN), lambda i: (i, 0)),
            full(NUM_G, FILTERS), full(1, FILTERS),
            full(FILTERS, FILTERS), full(1, FILTERS),
            full(FILTERS, HIDDEN), full(1, HIDDEN),
            full(HIDDEN, HIDDEN), full(1, HIDDEN),
            full(HIDDEN, FILTERS),
        ],
        out_specs=[
            pl.BlockSpec((BN, HIDDEN), lambda i: (i, 0)),
            pl.BlockSpec((BN, FILTERS), lambda i: (i, 0)),
        ],
        out_shape=[
            jax.ShapeDtypeStruct((N, HIDDEN), jnp.float32),
            jax.ShapeDtypeStruct((N, FILTERS), jnp.float32),
        ],
    )(g, dist, cfac, h, w1, b1, w2, b2, l2w, l2b, lw, lb, lin1n)


def _interact_final_body(g_ref, d_ref, c_ref, h_ref,
                         w1_ref, b1_ref, w2_ref, b2_ref,
                         l2w_ref, l2b_ref, lw_ref, lb_ref,
                         b_ref, o1w_ref, o1b_ref, o2w_ref, o2b_ref, acc_ref):
    i = pl.program_id(0)
    d3 = d_ref[...][:, :, None]
    cf3 = c_ref[...][:, :, None]
    offs = lax.broadcasted_iota(
        jnp.int32, (1, 1, NUM_G), 2).astype(jnp.float32) * _STEP
    rbf = jnp.exp(_COEFF * (d3 - offs) ** 2).reshape(EB, NUM_G)
    t = jnp.dot(rbf, w1_ref[...], preferred_element_type=jnp.float32) + b1_ref[...]
    t = _ssp(t)
    wf = jnp.dot(t, w2_ref[...], preferred_element_type=jnp.float32) + b2_ref[...]
    wf3 = wf.reshape(BN, MAXNB, FILTERS) * cf3
    msg = g_ref[...].reshape(BN, MAXNB, FILTERS) * wf3
    agg = jnp.sum(msg, axis=1)
    v = jnp.dot(agg, l2w_ref[...], preferred_element_type=jnp.float32) + l2b_ref[...]
    v = _ssp(v)
    v = jnp.dot(v, lw_ref[...], preferred_element_type=jnp.float32) + lb_ref[...]
    hn = h_ref[...] + v
    p = _ssp(jnp.dot(hn, o1w_ref[...],
                     preferred_element_type=jnp.float32) + o1b_ref[...])
    e = jnp.dot(p, o2w_ref[...], preferred_element_type=jnp.float32) + o2b_ref[...]
    ids = lax.broadcasted_iota(jnp.int32, (1, NUM_MOL), 1)
    moh = (b_ref[...] == ids).astype(jnp.float32)
    contrib = jnp.sum(moh * e, axis=0, keepdims=True)

    @pl.when(i == 0)
    def _():
        acc_ref[...] = jnp.zeros_like(acc_ref)

    acc_ref[...] += contrib


def _interact_final(g, dist, cfac, h, w1, b1, w2, b2, l2w, l2b, lw, lb,
                    batch2, o1w, o1b, o2w, o2b):
    full = lambda a, b: pl.BlockSpec((a, b), lambda i: (0, 0))
    return pl.pallas_call(
        _interact_final_body,
        grid=(NBLK,),
        in_specs=[
            pl.BlockSpec((EB, HIDDEN), lambda i: (i, 0)),
            pl.BlockSpec((BN, MAXNB), lambda i: (i, 0)),
            pl.BlockSpec((BN, MAXNB), lambda i: (i, 0)),
            pl.BlockSpec((BN, HIDDEN), lambda i: (i, 0)),
            full(NUM_G, FILTERS), full(1, FILTERS),
            full(FILTERS, FILTERS), full(1, FILTERS),
            full(FILTERS, HIDDEN), full(1, HIDDEN),
            full(HIDDEN, HIDDEN), full(1, HIDDEN),
            pl.BlockSpec((BN, 1), lambda i: (i, 0)),
            full(HIDDEN, HIDDEN // 2), full(1, HIDDEN // 2),
            full(HIDDEN // 2, 1), full(1, 1),
        ],
        out_specs=pl.BlockSpec((1, NUM_MOL), lambda i: (0, 0)),
        out_shape=jax.ShapeDtypeStruct((1, NUM_MOL), jnp.float32),
    )(g, dist, cfac, h, w1, b1, w2, b2, l2w, l2b, lw, lb,
      batch2, o1w, o1b, o2w, o2b)


def kernel(z, pos, batch, emb, mlp_w1, mlp_b1, mlp_w2, mlp_b2,
           conv_lin1_w, conv_lin2_w, conv_lin2_b, lin_w, lin_b,
           out1_w, out1_b, out2_w, out2_b):
    batch = batch.astype(jnp.int32)
    z = z.astype(jnp.int32)

    sq = jnp.sum(pos * pos, axis=1)
    posT_pad = jnp.pad(pos.T, ((0, 0), (0, NPAD - N)))
    sq_row = jnp.pad(sq, (0, NPAD - N))[None, :]
    batch_row = jnp.pad(batch, (0, NPAD - N),
                        constant_values=-2)[None, :]
    pos_pad = jnp.pad(pos, ((0, NPAD - N), (0, 0)))
    sq_col = jnp.pad(sq, (0, NPAD - N))[:, None]
    batch_col = jnp.pad(batch, (0, NPAD - N),
                        constant_values=-1)[:, None]

    first = batch[jnp.minimum(jnp.arange(RNB) * RB, N - 1)]
    ws = jnp.searchsorted(batch, first, side="left").astype(jnp.int32)
    ws = jnp.minimum((ws // 128) * 128, NPAD - W)
    ws3 = ws[:, None, None]

    src_t, dist_t, cfac_t = _radius_graph(ws3, posT_pad, sq_row, batch_row,
                                          pos_pad, sq_col, batch_col)
    src = src_t[:, :N].T
    dist = dist_t[:, :N].T
    cfac = cfac_t[:, :N].T
    idx_flat = src.reshape(E)

    h, xs = _embed(z[:, None], emb, conv_lin1_w[0])
    for l in range(NUM_INT - 1):
        g = _sc_gather(xs, idx_flat)
        h, xs = _interact(g, dist, cfac, h,
                          mlp_w1[l], mlp_b1[l][None, :],
                          mlp_w2[l], mlp_b2[l][None, :],
                          conv_lin2_w[l], conv_lin2_b[l][None, :],
                          lin_w[l], lin_b[l][None, :],
                          conv_lin1_w[l + 1])

    l = NUM_INT - 1
    g = _sc_gather(xs, idx_flat)
    out = _interact_final(g, dist, cfac, h,
                          mlp_w1[l], mlp_b1[l][None, :],
                          mlp_w2[l], mlp_b2[l][None, :],
                          conv_lin2_w[l], conv_lin2_b[l][None, :],
                          lin_w[l], lin_b[l][None, :],
                          batch[:, None], out1_w, out1_b[None, :],
                          out2_w, out2_b[None, :])
    return out.reshape(-1)


# speedup vs baseline: 1.2653x; 1.2653x over previous
import functools

import jax
import jax.numpy as jnp
import numpy as np
from jax import lax
from jax.experimental import pallas as pl
from jax.experimental.pallas import tpu as pltpu
from jax.experimental.pallas import tpu_sc as plsc

N = 10000
NUM_MOL = 100
HIDDEN = 128
FILTERS = 128
NUM_INT = 6
NUM_G = 50
CUTOFF = 10.0
MAXNB = 32

BN = 200
NBLK = N // BN
RB = 128
NPAD = 10240
RNB = NPAD // RB
W = 512
EB = BN * MAXNB
E = N * MAXNB

_OFFS = np.linspace(0.0, CUTOFF, NUM_G, dtype=np.float32)
_COEFF = np.float32(-0.5) / (_OFFS[1] - _OFFS[0]) ** 2
_STEP = np.float32(CUTOFF / (NUM_G - 1))
_LOG2 = np.float32(np.log(2.0))
_PI = np.float32(np.pi)


def _ssp(x):
    return jnp.maximum(x, 0.0) + jnp.log(1.0 + jnp.exp(-jnp.abs(x))) - _LOG2


def _radius_body(ws_ref, posT_ref, sqr_ref, brow_ref, pos_ref, sqc_ref,
                 bcol_ref, src_ref, dist_ref, c_ref):
    i = pl.program_id(0)
    ws = pl.multiple_of(ws_ref[0, 0], 128)
    pcT = posT_ref[...]
    posw = pos_ref[pl.ds(ws, W), :]
    sqw = sqc_ref[pl.ds(ws, W), :]
    bw = bcol_ref[pl.ds(ws, W), :]
    br = brow_ref[...]
    sqr = sqr_ref[...]
    d2 = sqw + sqr - 2.0 * jnp.dot(posw, pcT, preferred_element_type=jnp.float32)
    d2 = jnp.maximum(d2, 0.0)

    row_id = i * RB + lax.broadcasted_iota(jnp.int32, (1, RB), 1)
    col_id = ws + lax.broadcasted_iota(jnp.int32, (W, 1), 0)
    valid = (bw == br) & (col_id != row_id) & (d2 <= CUTOFF * CUTOFF)
    d2m = jnp.where(valid, d2, jnp.inf)

    ils = lax.broadcasted_iota(jnp.int32, (W, 1), 0).astype(jnp.float32)
    src_rows = []
    d2_rows = []
    for _ in range(MAXNB):
        m = jnp.min(d2m, axis=0, keepdims=True)
        eq = d2m == m
        jloc = jnp.min(jnp.where(eq, ils, np.float32(1e9)), axis=0,
                       keepdims=True)
        src_rows.append(ws + jloc.astype(jnp.int32))
        d2_rows.append(m)
        d2m = jnp.where(ils == jloc, jnp.inf, d2m)

    src_blk = jnp.concatenate(src_rows, axis=0)
    d2_blk = jnp.concatenate(d2_rows, axis=0)
    maskf = (d2_blk <= CUTOFF * CUTOFF).astype(jnp.float32)
    dist = jnp.sqrt(jnp.where(d2_blk <= CUTOFF * CUTOFF, d2_blk, 1.0))
    cfac = 0.5 * (jnp.cos(dist * _PI / CUTOFF) + 1.0) * maskf
    src_ref[...] = src_blk
    dist_ref[...] = dist
    c_ref[...] = cfac


def _radius_graph(ws3, posT_pad, sq_row, batch_row, pos_pad, sq_col,
                  batch_col):
    return pl.pallas_call(
        _radius_body,
        grid=(RNB,),
        in_specs=[
            pl.BlockSpec((None, 1, 1), lambda i: (i, 0, 0)),
            pl.BlockSpec((3, RB), lambda i: (0, i)),
            pl.BlockSpec((1, RB), lambda i: (0, i)),
            pl.BlockSpec((1, RB), lambda i: (0, i)),
            pl.BlockSpec((NPAD, 3), lambda i: (0, 0)),
            pl.BlockSpec((NPAD, 1), lambda i: (0, 0)),
            pl.BlockSpec((NPAD, 1), lambda i: (0, 0)),
        ],
        out_specs=[
            pl.BlockSpec((MAXNB, RB), lambda i: (0, i)),
            pl.BlockSpec((MAXNB, RB), lambda i: (0, i)),
            pl.BlockSpec((MAXNB, RB), lambda i: (0, i)),
        ],
        out_shape=[
            jax.ShapeDtypeStruct((MAXNB, NPAD), jnp.int32),
            jax.ShapeDtypeStruct((MAXNB, NPAD), jnp.float32),
            jax.ShapeDtypeStruct((MAXNB, NPAD), jnp.float32),
        ],
    )(ws3, posT_pad, sq_row, batch_row, pos_pad, sq_col, batch_col)


def _embed_body(z_ref, emb_ref, lin1_ref, h_ref, xs_ref):
    zb = z_ref[...]
    ids = lax.broadcasted_iota(jnp.int32, (1, 100), 1)
    oh = (zb == ids).astype(jnp.float32)
    h = jnp.dot(oh, emb_ref[...], preferred_element_type=jnp.float32)
    h_ref[...] = h
    xs_ref[...] = jnp.dot(h, lin1_ref[...], preferred_element_type=jnp.float32)


def _embed(z2, emb, lin1_0):
    return pl.pallas_call(
        _embed_body,
        grid=(NBLK,),
        in_specs=[
            pl.BlockSpec((BN, 1), lambda i: (i, 0)),
            pl.BlockSpec((100, HIDDEN), lambda i: (0, 0)),
            pl.BlockSpec((HIDDEN, FILTERS), lambda i: (0, 0)),
        ],
        out_specs=[
            pl.BlockSpec((BN, HIDDEN), lambda i: (i, 0)),
            pl.BlockSpec((BN, FILTERS), lambda i: (i, 0)),
        ],
        out_shape=[
            jax.ShapeDtypeStruct((N, HIDDEN), jnp.float32),
            jax.ShapeDtypeStruct((N, FILTERS), jnp.float32),
        ],
    )(z2, emb, lin1_0)


_NW = 32
_BPW = E // _NW
_CH = 200
_NCH = _BPW // _CH
_TW = 5632
_TOFF = 4608


def _sc_gather(xs, idx):
    mesh = plsc.VectorSubcoreMesh(core_axis_name="c", subcore_axis_name="s")

    @functools.partial(
        pl.kernel,
        out_type=jax.ShapeDtypeStruct((E, HIDDEN), jnp.float32),
        mesh=mesh,
        scratch_types=[
            pltpu.VMEM((_BPW,), jnp.int32),
            pltpu.VMEM((2, _CH, HIDDEN), jnp.float32),
            pltpu.VMEM_SHARED((_TW, HIDDEN), jnp.float32),
            pltpu.SemaphoreType.DMA,
            pltpu.SemaphoreType.DMA,
            pltpu.SemaphoreType.DMA,
            pltpu.SemaphoreType.DMA,
        ],
    )
    def k(table_hbm, idx_hbm, out_hbm, idx_v, rows_v, shared, sg0, sg1,
          so0, so1):
        wid = lax.axis_index("c") * 16 + lax.axis_index("s")
        base0 = wid * _BPW
        sg = (sg0, sg1)
        so = (so0, so1)

        @pl.when(lax.axis_index("s") == 0)
        def _():
            pltpu.sync_copy(
                table_hbm.at[pl.ds(lax.axis_index("c") * _TOFF, _TW)], shared)

        plsc.subcore_barrier()

        def gather_start(c, b):
            pltpu.async_copy(shared.at[idx_v.at[pl.ds(c * _CH, _CH)]],
                             rows_v.at[b], sg[b])

        def gather_wait(c, b):
            pltpu.make_async_copy(shared.at[idx_v.at[pl.ds(c * _CH, _CH)]],
                                  rows_v.at[b], sg[b]).wait()

        def out_start(c, b):
            pltpu.async_copy(rows_v.at[b],
                             out_hbm.at[pl.ds(base0 + c * _CH, _CH)], so[b])

        def out_wait(c, b):
            pltpu.make_async_copy(rows_v.at[b],
                                  out_hbm.at[pl.ds(base0 + c * _CH, _CH)],
                                  so[b]).wait()

        pltpu.sync_copy(idx_hbm.at[pl.ds(base0, _BPW)], idx_v)
        gather_start(0, 0)
        gather_wait(0, 0)
        out_start(0, 0)
        gather_start(1, 1)

        @pl.loop(1, _NCH - 1, step=2)
        def _(c):
            gather_wait(c, 1)
            out_start(c, 1)
            out_wait(c - 1, 0)
            gather_start(c + 1, 0)
            gather_wait(c + 1, 0)
            out_start(c + 1, 0)
            out_wait(c, 1)
            gather_start(c + 2, 1)

        c = _NCH - 1
        gather_wait(c, 1)
        out_start(c, 1)
        out_wait(c - 1, 0)
        out_wait(c, 1)

    xs_pad = jnp.pad(xs, ((0, _TOFF + _TW - N), (0, 0)))
    return k(xs_pad, idx)


def _interact_body(g_ref, d_ref, c_ref, h_ref,
                   w1_ref, b1_ref, w2_ref, b2_ref,
                   l2w_ref, l2b_ref, lw_ref, lb_ref, lin1n_ref,
                   h_out, xs_out):
    d3 = d_ref[...][:, :, None]
    cf3 = c_ref[...][:, :, None]
    offs = lax.broadcasted_iota(
        jnp.int32, (1, 1, NUM_G), 2).astype(jnp.float32) * _STEP
    rbf = jnp.exp(_COEFF * (d3 - offs) ** 2).reshape(EB, NUM_G)
    t = jnp.dot(rbf, w1_ref[...], preferred_element_type=jnp.float32) + b1_ref[...]
    t = _ssp(t)
    wf = jnp.dot(t, w2_ref[...], preferred_element_type=jnp.float32) + b2_ref[...]
    wf3 = wf.reshape(BN, MAXNB, FILTERS) * cf3
    msg = g_ref[...].reshape(BN, MAXNB, FILTERS) * wf3
    agg = jnp.sum(msg, axis=1)
    v = jnp.dot(agg, l2w_ref[...], preferred_element_type=jnp.float32) + l2b_ref[...]
    v = _ssp(v)
    v = jnp.dot(v, lw_ref[...], preferred_element_type=jnp.float32) + lb_ref[...]
    hn = h_ref[...] + v
    h_out[...] = hn
    xs_out[...] = jnp.dot(hn, lin1n_ref[...],
                          preferred_element_type=jnp.float32)


def _interact(g, dist, cfac, h, w1, b1, w2, b2, l2w, l2b, lw, lb, lin1n):
    full = lambda a, b: pl.BlockSpec((a, b), lambda i: (0, 0))
    return pl.pallas_call(
        _interact_body,
        grid=(NBLK,),
        in_specs=[
            pl.BlockSpec((EB, HIDDEN), lambda i: (i, 0)),
            pl.BlockSpec((BN, MAXNB), lambda i: (i, 0)),
            pl.BlockSpec((BN, MAXNB), lambda i: (i, 0)),
            pl.BlockSpec((BN, HIDDEN), lambda i: (i, 0)),
            full(NUM_G, FILTERS), full(1, FILTERS),
            full(FILTERS, FILTERS), full(1, FILTERS),
            full(FILTERS, HIDDEN), full(1, HIDDEN),
            full(HIDDEN, HIDDEN), full(1, HIDDEN),
            full(HIDDEN, FILTERS),
        ],
        out_specs=[
            pl.BlockSpec((BN, HIDDEN), lambda i: (i, 0)),
            pl.BlockSpec((BN, FILTERS), lambda i: (i, 0)),
        ],
        out_shape=[
            jax.ShapeDtypeStruct((N, HIDDEN), jnp.float32),
            jax.ShapeDtypeStruct((N, FILTERS), jnp.float32),
        ],
    )(g, dist, cfac, h, w1, b1, w2, b2, l2w, l2b, lw, lb, lin1n)


def _interact_final_body(g_ref, d_ref, c_ref, h_ref,
                         w1_ref, b1_ref, w2_ref, b2_ref,
                         l2w_ref, l2b_ref, lw_ref, lb_ref,
                         b_ref, o1w_ref, o1b_ref, o2w_ref, o2b_ref, acc_ref):
    i = pl.program_id(0)
    d3 = d_ref[...][:, :, None]
    cf3 = c_ref[...][:, :, None]
    offs = lax.broadcasted_iota(
        jnp.int32, (1, 1, NUM_G), 2).astype(jnp.float32) * _STEP
    rbf = jnp.exp(_COEFF * (d3 - offs) ** 2).reshape(EB, NUM_G)
    t = jnp.dot(rbf, w1_ref[...], preferred_element_type=jnp.float32) + b1_ref[...]
    t = _ssp(t)
    wf = jnp.dot(t, w2_ref[...], preferred_element_type=jnp.float32) + b2_ref[...]
    wf3 = wf.reshape(BN, MAXNB, FILTERS) * cf3
    msg = g_ref[...].reshape(BN, MAXNB, FILTERS) * wf3
    agg = jnp.sum(msg, axis=1)
    v = jnp.dot(agg, l2w_ref[...], preferred_element_type=jnp.float32) + l2b_ref[...]
    v = _ssp(v)
    v = jnp.dot(v, lw_ref[...], preferred_element_type=jnp.float32) + lb_ref[...]
    hn = h_ref[...] + v
    p = _ssp(jnp.dot(hn, o1w_ref[...],
                     preferred_element_type=jnp.float32) + o1b_ref[...])
    e = jnp.dot(p, o2w_ref[...], preferred_element_type=jnp.float32) + o2b_ref[...]
    ids = lax.broadcasted_iota(jnp.int32, (1, NUM_MOL), 1)
    moh = (b_ref[...] == ids).astype(jnp.float32)
    contrib = jnp.sum(moh * e, axis=0, keepdims=True)

    @pl.when(i == 0)
    def _():
        acc_ref[...] = jnp.zeros_like(acc_ref)

    acc_ref[...] += contrib


def _interact_final(g, dist, cfac, h, w1, b1, w2, b2, l2w, l2b, lw, lb,
                    batch2, o1w, o1b, o2w, o2b):
    full = lambda a, b: pl.BlockSpec((a, b), lambda i: (0, 0))
    return pl.pallas_call(
        _interact_final_body,
        grid=(NBLK,),
        in_specs=[
            pl.BlockSpec((EB, HIDDEN), lambda i: (i, 0)),
            pl.BlockSpec((BN, MAXNB), lambda i: (i, 0)),
            pl.BlockSpec((BN, MAXNB), lambda i: (i, 0)),
            pl.BlockSpec((BN, HIDDEN), lambda i: (i, 0)),
            full(NUM_G, FILTERS), full(1, FILTERS),
            full(FILTERS, FILTERS), full(1, FILTERS),
            full(FILTERS, HIDDEN), full(1, HIDDEN),
            full(HIDDEN, HIDDEN), full(1, HIDDEN),
            pl.BlockSpec((BN, 1), lambda i: (i, 0)),
            full(HIDDEN, HIDDEN // 2), full(1, HIDDEN // 2),
            full(HIDDEN // 2, 1), full(1, 1),
        ],
        out_specs=pl.BlockSpec((1, NUM_MOL), lambda i: (0, 0)),
        out_shape=jax.ShapeDtypeStruct((1, NUM_MOL), jnp.float32),
    )(g, dist, cfac, h, w1, b1, w2, b2, l2w, l2b, lw, lb,
      batch2, o1w, o1b, o2w, o2b)


def kernel(z, pos, batch, emb, mlp_w1, mlp_b1, mlp_w2, mlp_b2,
           conv_lin1_w, conv_lin2_w, conv_lin2_b, lin_w, lin_b,
           out1_w, out1_b, out2_w, out2_b):
    batch = batch.astype(jnp.int32)
    z = z.astype(jnp.int32)

    sq = jnp.sum(pos * pos, axis=1)
    posT_pad = jnp.pad(pos.T, ((0, 0), (0, NPAD - N)))
    sq_row = jnp.pad(sq, (0, NPAD - N))[None, :]
    batch_row = jnp.pad(batch, (0, NPAD - N),
                        constant_values=-2)[None, :]
    pos_pad = jnp.pad(pos, ((0, NPAD - N), (0, 0)))
    sq_col = jnp.pad(sq, (0, NPAD - N))[:, None]
    batch_col = jnp.pad(batch, (0, NPAD - N),
                        constant_values=-1)[:, None]

    first = batch[jnp.minimum(jnp.arange(RNB) * RB, N - 1)]
    ws = jnp.searchsorted(batch, first, side="left").astype(jnp.int32)
    ws = jnp.minimum((ws // 128) * 128, NPAD - W)
    ws3 = ws[:, None, None]

    src_t, dist_t, cfac_t = _radius_graph(ws3, posT_pad, sq_row, batch_row,
                                          pos_pad, sq_col, batch_col)
    src = src_t[:, :N].T
    dist = dist_t[:, :N].T
    cfac = cfac_t[:, :N].T
    idx_flat = src.reshape(E)
    idx_flat = jnp.where(jnp.arange(E) < E // 2, idx_flat,
                         idx_flat - _TOFF)

    h, xs = _embed(z[:, None], emb, conv_lin1_w[0])
    for l in range(NUM_INT - 1):
        g = _sc_gather(xs, idx_flat)
        h, xs = _interact(g, dist, cfac, h,
                          mlp_w1[l], mlp_b1[l][None, :],
                          mlp_w2[l], mlp_b2[l][None, :],
                          conv_lin2_w[l], conv_lin2_b[l][None, :],
                          lin_w[l], lin_b[l][None, :],
                          conv_lin1_w[l + 1])

    l = NUM_INT - 1
    g = _sc_gather(xs, idx_flat)
    out = _interact_final(g, dist, cfac, h,
                          mlp_w1[l], mlp_b1[l][None, :],
                          mlp_w2[l], mlp_b2[l][None, :],
                          conv_lin2_w[l], conv_lin2_b[l][None, :],
                          lin_w[l], lin_b[l][None, :],
                          batch[:, None], out1_w, out1_b[None, :],
                          out2_w, out2_b[None, :])
    return out.reshape(-1)
